# baseline (speedup 1.0000x reference)
import functools

import jax
import jax.numpy as jnp
from jax.experimental import pallas as pl

_LATENT_DIM = 64
_NUM_CODES = 1024
_BLOCK_ROWS = 8192
_SUB_ROWS = 128


def _vq_body(x_ref, emb_ref, o_ref):
    emb = emb_ref[...]
    e2 = jnp.sum(emb * emb, axis=0, keepdims=True)
    for k in range(_BLOCK_ROWS // _SUB_ROWS):
        xb = x_ref[pl.ds(k * _SUB_ROWS, _SUB_ROWS), :]
        sim = jnp.dot(xb, emb, preferred_element_type=jnp.float32)
        scores = e2 - 2.0 * sim
        idx = jnp.argmin(scores, axis=1).reshape(-1, 1)
        col = jax.lax.broadcasted_iota(jnp.int32, scores.shape, 1)
        onehot = (col == idx).astype(jnp.float32)
        o_ref[pl.ds(k * _SUB_ROWS, _SUB_ROWS), :] = jax.lax.dot_general(
            onehot, emb, (((1,), (1,)), ((), ())),
            preferred_element_type=jnp.float32)


@functools.partial(jax.jit, static_argnames=("interpret",))
def kernel(x, embeddings, interpret=False):
    orig_shape = x.shape
    xf = x.reshape(-1, _LATENT_DIM)
    rows = xf.shape[0]
    grid = (rows // _BLOCK_ROWS,)
    out = pl.pallas_call(
        _vq_body,
        grid=grid,
        in_specs=[
            pl.BlockSpec((_BLOCK_ROWS, _LATENT_DIM), lambda i: (i, 0)),
            pl.BlockSpec((_LATENT_DIM, _NUM_CODES), lambda i: (0, 0)),
        ],
        out_specs=pl.BlockSpec((_BLOCK_ROWS, _LATENT_DIM), lambda i: (i, 0)),
        out_shape=jax.ShapeDtypeStruct((rows, _LATENT_DIM), jnp.float32),
        interpret=interpret,
    )(xf, embeddings)
    return out.reshape(orig_shape)


# speedup vs baseline: 1.0050x; 1.0050x over previous
import functools

import jax
import jax.numpy as jnp
from jax.experimental import pallas as pl

_LATENT_DIM = 64
_NUM_CODES = 1024
_BLOCK_ROWS = 8192
_SUB_ROWS = 256


def _vq_body(x_ref, emb_ref, o_ref):
    emb = emb_ref[...]
    e2 = jnp.sum(emb * emb, axis=0, keepdims=True)
    for k in range(_BLOCK_ROWS // _SUB_ROWS):
        xb = x_ref[pl.ds(k * _SUB_ROWS, _SUB_ROWS), :]
        sim = jnp.dot(xb.astype(jnp.bfloat16), emb.astype(jnp.bfloat16),
                      preferred_element_type=jnp.float32)
        scores = e2 - 2.0 * sim
        idx = jnp.argmin(scores, axis=1).reshape(-1, 1)
        col = jax.lax.broadcasted_iota(jnp.int32, scores.shape, 1)
        onehot = (col == idx).astype(jnp.bfloat16)
        o_ref[pl.ds(k * _SUB_ROWS, _SUB_ROWS), :] = jax.lax.dot_general(
            onehot, emb.astype(jnp.bfloat16), (((1,), (1,)), ((), ())),
            preferred_element_type=jnp.float32)


@functools.partial(jax.jit, static_argnames=("interpret",))
def kernel(x, embeddings, interpret=False):
    orig_shape = x.shape
    xf = x.reshape(-1, _LATENT_DIM)
    rows = xf.shape[0]
    grid = (rows // _BLOCK_ROWS,)
    out = pl.pallas_call(
        _vq_body,
        grid=grid,
        in_specs=[
            pl.BlockSpec((_BLOCK_ROWS, _LATENT_DIM), lambda i: (i, 0)),
            pl.BlockSpec((_LATENT_DIM, _NUM_CODES), lambda i: (0, 0)),
        ],
        out_specs=pl.BlockSpec((_BLOCK_ROWS, _LATENT_DIM), lambda i: (i, 0)),
        out_shape=jax.ShapeDtypeStruct((rows, _LATENT_DIM), jnp.float32),
        interpret=interpret,
    )(xf, embeddings)
    return out.reshape(orig_shape)


# speedup vs baseline: 1.0098x; 1.0048x over previous
import functools

import jax
import jax.numpy as jnp
from jax.experimental import pallas as pl

_LATENT_DIM = 64
_NUM_CODES = 1024
_BLOCK_ROWS = 8192
_SUB_ROWS = 256


def _vq_body(x_ref, emb_ref, o_ref):
    emb = emb_ref[...]
    e2 = jnp.sum(emb * emb, axis=0, keepdims=True)
    for k in range(_BLOCK_ROWS // _SUB_ROWS):
        xb = x_ref[pl.ds(k * _SUB_ROWS, _SUB_ROWS), :]
        sim = jnp.dot(xb, emb, preferred_element_type=jnp.float32)
        scores = e2 - 2.0 * sim
        idx = jnp.argmin(scores, axis=1).reshape(-1, 1)
        col = jax.lax.broadcasted_iota(jnp.int32, scores.shape, 1)
        onehot = (col == idx).astype(jnp.float32)
        o_ref[pl.ds(k * _SUB_ROWS, _SUB_ROWS), :] = jax.lax.dot_general(
            onehot, emb, (((1,), (1,)), ((), ())),
            preferred_element_type=jnp.float32)


@functools.partial(jax.jit, static_argnames=("interpret",))
def kernel(x, embeddings, interpret=False):
    orig_shape = x.shape
    xf = x.reshape(-1, _LATENT_DIM)
    rows = xf.shape[0]
    grid = (rows // _BLOCK_ROWS,)
    out = pl.pallas_call(
        _vq_body,
        grid=grid,
        in_specs=[
            pl.BlockSpec((_BLOCK_ROWS, _LATENT_DIM), lambda i: (i, 0)),
            pl.BlockSpec((_LATENT_DIM, _NUM_CODES), lambda i: (0, 0)),
        ],
        out_specs=pl.BlockSpec((_BLOCK_ROWS, _LATENT_DIM), lambda i: (i, 0)),
        out_shape=jax.ShapeDtypeStruct((rows, _LATENT_DIM), jnp.float32),
        interpret=interpret,
    )(xf, embeddings)
    return out.reshape(orig_shape)
